# Initial kernel scaffold; baseline (speedup 1.0000x reference)
import jax
import jax.numpy as jnp
from jax.experimental import pallas as pl


def kernel(x, a_indices, a_values):
    raise NotImplementedError("write your pallas kernel here")


# speedup vs baseline: 3.6239x; 3.6239x over previous
import functools

import jax
import jax.numpy as jnp
from jax import lax
from jax.experimental import pallas as pl
from jax.experimental.pallas import tpu as pltpu
from jax.experimental.pallas import tpu_sc as plsc

N_NODES = 10000
D_FEAT = 128
N_EDGES = 320000

NC = 2
NS = 16
NVREG = D_FEAT // 16
CHUNK = 128
NCHUNK = 80
NB = 16
NBLK = NCHUNK // NB
EPT = NCHUNK * CHUNK
E_PAD = NC * NS * EPT
GPB = 16
NGRP = CHUNK // GPB
N_PAD = 10240
RPT = N_PAD // NS
ZB = RPT // CHUNK

_mesh = plsc.VectorSubcoreMesh(core_axis_name="c", subcore_axis_name="s")


@functools.partial(
    pl.kernel,
    out_type=jax.ShapeDtypeStruct((NC, N_PAD, D_FEAT), jnp.float32),
    mesh=_mesh,
    scratch_types=[
        pltpu.VMEM_SHARED((N_PAD, D_FEAT), jnp.float32),
        pltpu.VMEM((NB, CHUNK), jnp.int32),
        pltpu.VMEM((NB, CHUNK), jnp.int32),
        pltpu.VMEM((NB, CHUNK), jnp.float32),
        pltpu.VMEM((CHUNK, D_FEAT), jnp.float32),
        pltpu.SemaphoreType.DMA,
    ],
)
def _mp_sc_kernel(x_hbm, idxj_hbm, idxi_hbm, val_hbm, out_hbm,
                  acc, idxj_v, idxi_v, val_v, rows_v, sem):
    c = lax.axis_index("c")
    s = lax.axis_index("s")
    row0 = s * RPT

    zeros16 = jnp.zeros((16,), jnp.float32)

    def zero_row(r, carry):
        for q in range(NVREG):
            rows_v[r, pl.ds(q * 16, 16)] = zeros16
        return carry

    lax.fori_loop(0, CHUNK, zero_row, 0)
    for b in range(ZB):
        pltpu.sync_copy(rows_v, acc.at[pl.ds(row0 + b * CHUNK, CHUNK)])
    plsc.subcore_barrier()

    def block_body(b, carry):
        pltpu.sync_copy(idxj_hbm.at[c].at[s].at[b], idxj_v)
        pltpu.sync_copy(idxi_hbm.at[c].at[s].at[b], idxi_v)
        pltpu.sync_copy(val_hbm.at[c].at[s].at[b], val_v)

        def chunk_body(k, carry1):
            pltpu.async_copy(x_hbm.at[idxj_v.at[k]], rows_v, sem).wait()

            def group_body(g, carry2):
                v16 = val_v[k, pl.ds(g * GPB, GPB)]
                for r in range(GPB):
                    vvec = jnp.full((16,), v16[r], jnp.float32)
                    row = g * GPB + r
                    for q in range(NVREG):
                        rows_v[row, pl.ds(q * 16, 16)] = (
                            rows_v[row, pl.ds(q * 16, 16)] * vvec)
                return carry2

            lax.fori_loop(0, NGRP, group_body, 0)
            pltpu.sync_copy(rows_v, acc.at[idxi_v.at[k]], add=True)
            return carry1

        lax.fori_loop(0, NB, chunk_body, 0)
        return carry

    lax.fori_loop(0, NBLK, block_body, 0)
    plsc.subcore_barrier()

    pltpu.sync_copy(acc.at[pl.ds(row0, RPT)], out_hbm.at[c].at[pl.ds(row0, RPT)])


def _combine_body(p_ref, o_ref):
    o_ref[...] = p_ref[0] + p_ref[1]


_N_BLK = 8


def _combine(partials):
    return pl.pallas_call(
        _combine_body,
        out_shape=jax.ShapeDtypeStruct((N_PAD, D_FEAT), jnp.float32),
        grid=(_N_BLK,),
        in_specs=[pl.BlockSpec((NC, N_PAD // _N_BLK, D_FEAT),
                               lambda i: (0, i, 0))],
        out_specs=pl.BlockSpec((N_PAD // _N_BLK, D_FEAT), lambda i: (i, 0)),
    )(partials)


def kernel(x, a_indices, a_values):
    pad = E_PAD - N_EDGES
    idx_i = jnp.pad(a_indices[0].astype(jnp.int32), (0, pad))
    idx_j = jnp.pad(a_indices[1].astype(jnp.int32), (0, pad))
    vals = jnp.pad(a_values.astype(jnp.float32), (0, pad))
    idx_i = idx_i.reshape(NC, NS, NBLK, NB, CHUNK)
    idx_j = idx_j.reshape(NC, NS, NBLK, NB, CHUNK)
    vals = vals.reshape(NC, NS, NBLK, NB, CHUNK)
    x_pad = jnp.pad(x, ((0, N_PAD - N_NODES), (0, 0)))
    partials = _mp_sc_kernel(x_pad, idx_j, idx_i, vals)
    return _combine(partials)[:N_NODES]


# speedup vs baseline: 4.3873x; 1.2106x over previous
import functools

import jax
import jax.numpy as jnp
from jax import lax
from jax.experimental import pallas as pl
from jax.experimental.pallas import tpu as pltpu
from jax.experimental.pallas import tpu_sc as plsc

N_NODES = 10000
D_FEAT = 128
N_EDGES = 320000

NC = 2
NS = 16
NVREG = D_FEAT // 16
CHUNK = 128
NCHUNK = 80
NB = 16
NBLK = NCHUNK // NB
EPT = NCHUNK * CHUNK
E_PAD = NC * NS * EPT
GPB = 16
NGRP = CHUNK // GPB
N_PAD = 10240
RPT = N_PAD // NS
ZB = RPT // CHUNK

_mesh = plsc.VectorSubcoreMesh(core_axis_name="c", subcore_axis_name="s")


@functools.partial(
    pl.kernel,
    out_type=jax.ShapeDtypeStruct((NC, N_PAD, D_FEAT), jnp.float32),
    mesh=_mesh,
    scratch_types=[
        pltpu.VMEM_SHARED((N_PAD, D_FEAT), jnp.float32),
        pltpu.VMEM((NB, CHUNK), jnp.int32),
        pltpu.VMEM((NB, CHUNK), jnp.int32),
        pltpu.VMEM((NB, CHUNK), jnp.float32),
        pltpu.VMEM((CHUNK, D_FEAT), jnp.float32),
        pltpu.VMEM((CHUNK, D_FEAT), jnp.float32),
        pltpu.SemaphoreType.DMA,
        pltpu.SemaphoreType.DMA,
    ],
)
def _mp_sc_kernel(x_hbm, idxj_hbm, idxi_hbm, val_hbm, out_hbm,
                  acc, idxj_v, idxi_v, val_v, rows_a, rows_b, gsem_a, gsem_b):
    c = lax.axis_index("c")
    s = lax.axis_index("s")
    row0 = s * RPT

    zeros16 = jnp.zeros((16,), jnp.float32)

    def zero_row(r, carry):
        for q in range(NVREG):
            rows_a[r, pl.ds(q * 16, 16)] = zeros16
        return carry

    lax.fori_loop(0, CHUNK, zero_row, 0)
    for b in range(ZB):
        pltpu.sync_copy(rows_a, acc.at[pl.ds(row0 + b * CHUNK, CHUNK)])
    plsc.subcore_barrier()

    def scale_rows(rows_ref, k):
        def group_body(g, carry2):
            v16 = val_v[k, pl.ds(g * GPB, GPB)]
            for r in range(GPB):
                vvec = jnp.full((16,), v16[r], jnp.float32)
                row = g * GPB + r
                for q in range(NVREG):
                    rows_ref[row, pl.ds(q * 16, 16)] = (
                        rows_ref[row, pl.ds(q * 16, 16)] * vvec)
            return carry2

        lax.fori_loop(0, NGRP, group_body, 0)

    def wait_gather(rows_ref, gsem):
        pltpu.make_async_copy(x_hbm.at[pl.ds(0, CHUNK)], rows_ref, gsem).wait()

    def block_body(b, carry):
        pltpu.sync_copy(idxj_hbm.at[c].at[s].at[b], idxj_v)
        pltpu.sync_copy(idxi_hbm.at[c].at[s].at[b], idxi_v)
        pltpu.sync_copy(val_hbm.at[c].at[s].at[b], val_v)
        pltpu.async_copy(x_hbm.at[idxj_v.at[0]], rows_a, gsem_a)
        pltpu.async_copy(x_hbm.at[idxj_v.at[1]], rows_b, gsem_b)

        def pair_body(p, carry1):
            for (rows_ref, gsem, k) in ((rows_a, gsem_a, 2 * p),
                                        (rows_b, gsem_b, 2 * p + 1)):
                wait_gather(rows_ref, gsem)
                scale_rows(rows_ref, k)
                pltpu.sync_copy(rows_ref, acc.at[idxi_v.at[k]], add=True)
                pltpu.async_copy(x_hbm.at[idxj_v.at[k + 2]], rows_ref, gsem)
            return carry1

        lax.fori_loop(0, NB // 2 - 1, pair_body, 0)
        for (rows_ref, gsem, k) in ((rows_a, gsem_a, NB - 2),
                                    (rows_b, gsem_b, NB - 1)):
            wait_gather(rows_ref, gsem)
            scale_rows(rows_ref, k)
            pltpu.sync_copy(rows_ref, acc.at[idxi_v.at[k]], add=True)
        return carry

    lax.fori_loop(0, NBLK, block_body, 0)
    plsc.subcore_barrier()

    pltpu.sync_copy(acc.at[pl.ds(row0, RPT)], out_hbm.at[c].at[pl.ds(row0, RPT)])


def _combine_body(p_ref, o_ref):
    o_ref[...] = p_ref[0] + p_ref[1]


_N_BLK = 8


def _combine(partials):
    return pl.pallas_call(
        _combine_body,
        out_shape=jax.ShapeDtypeStruct((N_PAD, D_FEAT), jnp.float32),
        grid=(_N_BLK,),
        in_specs=[pl.BlockSpec((NC, N_PAD // _N_BLK, D_FEAT),
                               lambda i: (0, i, 0))],
        out_specs=pl.BlockSpec((N_PAD // _N_BLK, D_FEAT), lambda i: (i, 0)),
    )(partials)


def kernel(x, a_indices, a_values):
    pad = E_PAD - N_EDGES
    idx_i = jnp.pad(a_indices[0].astype(jnp.int32), (0, pad))
    idx_j = jnp.pad(a_indices[1].astype(jnp.int32), (0, pad))
    vals = jnp.pad(a_values.astype(jnp.float32), (0, pad))
    idx_i = idx_i.reshape(NC, NS, NBLK, NB, CHUNK)
    idx_j = idx_j.reshape(NC, NS, NBLK, NB, CHUNK)
    vals = vals.reshape(NC, NS, NBLK, NB, CHUNK)
    x_pad = jnp.pad(x, ((0, N_PAD - N_NODES), (0, 0)))
    partials = _mp_sc_kernel(x_pad, idx_j, idx_i, vals)
    return _combine(partials)[:N_NODES]
